# initial kernel scaffold (unmeasured)
import jax
import jax.numpy as jnp
from jax import lax
from jax.experimental import pallas as pl
from jax.experimental.pallas import tpu as pltpu


def kernel(
    x,
):
    def body(*refs):
        pass

    out_shape = jax.ShapeDtypeStruct(..., jnp.float32)
    return pl.pallas_call(body, out_shape=out_shape)(...)


# baseline (device time: 240622 ns/iter reference)
import jax
import jax.numpy as jnp
from jax import lax
from jax.experimental import pallas as pl
from jax.experimental.pallas import tpu as pltpu

N_DEV = 16


def kernel(x):
    m, n = x.shape
    mc = m // N_DEV
    xr = x.reshape(N_DEV, mc, n).astype(jnp.bfloat16)

    def body(x_ref, out_ref, sbuf, rbuf, ssem, rsem, ag_rsem):
        my = lax.axis_index("i")
        right = lax.rem(my + 1, N_DEV)
        left = lax.rem(my + N_DEV - 1, N_DEV)

        bar = pltpu.get_barrier_semaphore()
        for nbr in (left, right):
            pl.semaphore_signal(
                bar, inc=1, device_id=(nbr,),
                device_id_type=pl.DeviceIdType.MESH,
            )
        pl.semaphore_wait(bar, 2)

        for s in range(N_DEV - 1):
            c_send = lax.rem(my - s + 2 * N_DEV, N_DEV)
            if s == 0:
                sbuf[0] = x_ref[c_send]
            else:
                sbuf[0] = rbuf[s - 1] + x_ref[c_send]
            rdma = pltpu.make_async_remote_copy(
                src_ref=sbuf.at[0],
                dst_ref=rbuf.at[s],
                send_sem=ssem.at[s],
                recv_sem=rsem.at[s],
                device_id=(right,),
                device_id_type=pl.DeviceIdType.MESH,
            )
            rdma.start()
            rdma.wait()

        c_mine = lax.rem(my + 1, N_DEV)
        out_ref[c_mine] = rbuf[N_DEV - 2] + x_ref[c_mine]

        for t in range(N_DEV - 1):
            c = lax.rem(my + 1 - t + 2 * N_DEV, N_DEV)
            rdma = pltpu.make_async_remote_copy(
                src_ref=out_ref.at[c],
                dst_ref=out_ref.at[c],
                send_sem=ssem.at[t],
                recv_sem=ag_rsem.at[t],
                device_id=(right,),
                device_id_type=pl.DeviceIdType.MESH,
            )
            rdma.start()
            rdma.wait()

    out = pl.pallas_call(
        body,
        out_shape=jax.ShapeDtypeStruct((N_DEV, mc, n), jnp.bfloat16),
        in_specs=[pl.BlockSpec(memory_space=pltpu.VMEM)],
        out_specs=pl.BlockSpec(memory_space=pltpu.VMEM),
        scratch_shapes=[
            pltpu.VMEM((1, mc, n), jnp.bfloat16),
            pltpu.VMEM((N_DEV - 1, mc, n), jnp.bfloat16),
            pltpu.SemaphoreType.DMA((N_DEV - 1,)),
            pltpu.SemaphoreType.DMA((N_DEV - 1,)),
            pltpu.SemaphoreType.DMA((N_DEV - 1,)),
        ],
        compiler_params=pltpu.CompilerParams(collective_id=0),
    )(xr)
    return out.reshape(m, n).astype(jnp.float32)


# device time: 196680 ns/iter; 1.2234x vs baseline; 1.2234x over previous
import jax
import jax.numpy as jnp
from jax import lax
from jax.experimental import pallas as pl
from jax.experimental.pallas import tpu as pltpu

N_DEV = 16


def kernel(x):
    m, n = x.shape
    mc = m // (2 * N_DEV)
    xr = x.reshape(2 * N_DEV, mc, n).astype(jnp.bfloat16)

    def body(x_ref, out_ref, sbuf_a, sbuf_b, rbuf_a, rbuf_b,
             ssem_a, ssem_b, rsem_a, rsem_b, ag_rsem_a, ag_rsem_b):
        my = lax.axis_index("i")
        right = lax.rem(my + 1, N_DEV)
        left = lax.rem(my + N_DEV - 1, N_DEV)

        bar = pltpu.get_barrier_semaphore()
        for nbr in (left, right):
            pl.semaphore_signal(
                bar, inc=1, device_id=(nbr,),
                device_id_type=pl.DeviceIdType.MESH,
            )
        pl.semaphore_wait(bar, 2)

        for s in range(N_DEV - 1):
            ca = lax.rem(my - s + 2 * N_DEV, N_DEV)
            cb = lax.rem(my + s, N_DEV)
            if s == 0:
                sbuf_a[0] = x_ref[ca]
                sbuf_b[0] = x_ref[N_DEV + cb]
            else:
                sbuf_a[0] = rbuf_a[s - 1] + x_ref[ca]
                sbuf_b[0] = rbuf_b[s - 1] + x_ref[N_DEV + cb]
            rdma_a = pltpu.make_async_remote_copy(
                src_ref=sbuf_a.at[0],
                dst_ref=rbuf_a.at[s],
                send_sem=ssem_a.at[s],
                recv_sem=rsem_a.at[s],
                device_id=(right,),
                device_id_type=pl.DeviceIdType.MESH,
            )
            rdma_b = pltpu.make_async_remote_copy(
                src_ref=sbuf_b.at[0],
                dst_ref=rbuf_b.at[s],
                send_sem=ssem_b.at[s],
                recv_sem=rsem_b.at[s],
                device_id=(left,),
                device_id_type=pl.DeviceIdType.MESH,
            )
            rdma_a.start()
            rdma_b.start()
            rdma_a.wait()
            rdma_b.wait()

        c_mine_a = lax.rem(my + 1, N_DEV)
        c_mine_b = lax.rem(my + N_DEV - 1, N_DEV)
        out_ref[c_mine_a] = rbuf_a[N_DEV - 2] + x_ref[c_mine_a]
        out_ref[N_DEV + c_mine_b] = rbuf_b[N_DEV - 2] + x_ref[N_DEV + c_mine_b]

        for t in range(N_DEV - 1):
            ca = lax.rem(my + 1 - t + 2 * N_DEV, N_DEV)
            cb = lax.rem(my - 1 + t + 2 * N_DEV, N_DEV)
            rdma_a = pltpu.make_async_remote_copy(
                src_ref=out_ref.at[ca],
                dst_ref=out_ref.at[ca],
                send_sem=ssem_a.at[t],
                recv_sem=ag_rsem_a.at[t],
                device_id=(right,),
                device_id_type=pl.DeviceIdType.MESH,
            )
            rdma_b = pltpu.make_async_remote_copy(
                src_ref=out_ref.at[N_DEV + cb],
                dst_ref=out_ref.at[N_DEV + cb],
                send_sem=ssem_b.at[t],
                recv_sem=ag_rsem_b.at[t],
                device_id=(left,),
                device_id_type=pl.DeviceIdType.MESH,
            )
            rdma_a.start()
            rdma_b.start()
            rdma_a.wait()
            rdma_b.wait()

    out = pl.pallas_call(
        body,
        out_shape=jax.ShapeDtypeStruct((2 * N_DEV, mc, n), jnp.bfloat16),
        in_specs=[pl.BlockSpec(memory_space=pltpu.VMEM)],
        out_specs=pl.BlockSpec(memory_space=pltpu.VMEM),
        scratch_shapes=[
            pltpu.VMEM((1, mc, n), jnp.bfloat16),
            pltpu.VMEM((1, mc, n), jnp.bfloat16),
            pltpu.VMEM((N_DEV - 1, mc, n), jnp.bfloat16),
            pltpu.VMEM((N_DEV - 1, mc, n), jnp.bfloat16),
            pltpu.SemaphoreType.DMA((N_DEV - 1,)),
            pltpu.SemaphoreType.DMA((N_DEV - 1,)),
            pltpu.SemaphoreType.DMA((N_DEV - 1,)),
            pltpu.SemaphoreType.DMA((N_DEV - 1,)),
            pltpu.SemaphoreType.DMA((N_DEV - 1,)),
            pltpu.SemaphoreType.DMA((N_DEV - 1,)),
        ],
        compiler_params=pltpu.CompilerParams(collective_id=0),
    )(xr)
    return out.reshape(m, n).astype(jnp.float32)


# device time: 125673 ns/iter; 1.9147x vs baseline; 1.5650x over previous
import jax
import jax.numpy as jnp
from jax import lax
from jax.experimental import pallas as pl
from jax.experimental.pallas import tpu as pltpu

N_DEV = 16
NSUB = 4


def kernel(x):
    m, n = x.shape
    mc = m // (2 * N_DEV)
    sub = mc // NSUB
    xr = x.reshape(2 * N_DEV, mc, n).astype(jnp.bfloat16)

    def body(x_ref, out_ref, rbuf_a, rbuf_b,
             ssem_a, ssem_b, rsem_a, rsem_b,
             ag_ssem_a, ag_ssem_b, ag_rsem_a, ag_rsem_b):
        my = lax.axis_index("i")
        right = lax.rem(my + 1, N_DEV)
        left = lax.rem(my + N_DEV - 1, N_DEV)

        bar = pltpu.get_barrier_semaphore()
        for nbr in (left, right):
            pl.semaphore_signal(
                bar, inc=1, device_id=(nbr,),
                device_id_type=pl.DeviceIdType.MESH,
            )
        pl.semaphore_wait(bar, 2)

        rows = [pl.ds(k * sub, sub) for k in range(NSUB)]
        rs_descs = {"a": [], "b": []}
        ag_descs = {"a": [], "b": []}

        def rs_desc(d, s, k, src):
            rbuf, rsem, ssem, peer = (
                (rbuf_a, rsem_a, ssem_a, right) if d == "a"
                else (rbuf_b, rsem_b, ssem_b, left)
            )
            return pltpu.make_async_remote_copy(
                src_ref=src,
                dst_ref=rbuf.at[s, rows[k]],
                send_sem=ssem.at[s, k],
                recv_sem=rsem.at[s, k],
                device_id=(peer,),
                device_id_type=pl.DeviceIdType.MESH,
            )

        def ag_desc(d, t, k, c):
            ssem, rsem, peer = (
                (ag_ssem_a, ag_rsem_a, right) if d == "a"
                else (ag_ssem_b, ag_rsem_b, left)
            )
            return pltpu.make_async_remote_copy(
                src_ref=out_ref.at[c, rows[k]],
                dst_ref=out_ref.at[c, rows[k]],
                send_sem=ssem.at[t, k],
                recv_sem=rsem.at[t, k],
                device_id=(peer,),
                device_id_type=pl.DeviceIdType.MESH,
            )

        for s in range(N_DEV - 1):
            ca = lax.rem(my - s + 2 * N_DEV, N_DEV)
            cb = N_DEV + lax.rem(my + s, N_DEV)
            hop_a, hop_b = [], []
            for k in range(NSUB):
                for d, c, rbuf, hop in (
                    ("a", ca, rbuf_a, hop_a), ("b", cb, rbuf_b, hop_b),
                ):
                    if s == 0:
                        src = x_ref.at[c, rows[k]]
                    else:
                        rs_descs[d][s - 1][k].wait_recv()
                        if s >= 2:
                            rs_descs[d][s - 2][k].wait_send()
                        rbuf[s - 1, rows[k]] = (
                            rbuf[s - 1, rows[k]] + x_ref[c, rows[k]]
                        )
                        src = rbuf.at[s - 1, rows[k]]
                    desc = rs_desc(d, s, k, src)
                    desc.start()
                    hop.append(desc)
            rs_descs["a"].append(hop_a)
            rs_descs["b"].append(hop_b)

        c_mine_a = lax.rem(my + 1, N_DEV)
        c_mine_b = N_DEV + lax.rem(my + N_DEV - 1, N_DEV)

        for t in range(N_DEV - 1):
            ca = lax.rem(my + 1 - t + 2 * N_DEV, N_DEV)
            cb = N_DEV + lax.rem(my - 1 + t + 2 * N_DEV, N_DEV)
            hop_a, hop_b = [], []
            for k in range(NSUB):
                for d, c, c_mine, rbuf, hop in (
                    ("a", ca, c_mine_a, rbuf_a, hop_a),
                    ("b", cb, c_mine_b, rbuf_b, hop_b),
                ):
                    if t == 0:
                        rs_descs[d][N_DEV - 2][k].wait_recv()
                        rs_descs[d][N_DEV - 3][k].wait_send()
                        out_ref[c_mine, rows[k]] = (
                            rbuf[N_DEV - 2, rows[k]] + x_ref[c_mine, rows[k]]
                        )
                    else:
                        ag_descs[d][t - 1][k].wait_recv()
                        if t == 1:
                            rs_descs[d][N_DEV - 2][k].wait_send()
                        else:
                            ag_descs[d][t - 2][k].wait_send()
                    desc = ag_desc(d, t, k, c)
                    desc.start()
                    hop.append(desc)
            ag_descs["a"].append(hop_a)
            ag_descs["b"].append(hop_b)

        for d in ("a", "b"):
            for k in range(NSUB):
                ag_descs[d][N_DEV - 2][k].wait_recv()
                ag_descs[d][N_DEV - 3][k].wait_send()
                ag_descs[d][N_DEV - 2][k].wait_send()

    out = pl.pallas_call(
        body,
        out_shape=jax.ShapeDtypeStruct((2 * N_DEV, mc, n), jnp.bfloat16),
        in_specs=[pl.BlockSpec(memory_space=pltpu.VMEM)],
        out_specs=pl.BlockSpec(memory_space=pltpu.VMEM),
        scratch_shapes=[
            pltpu.VMEM((N_DEV - 1, mc, n), jnp.bfloat16),
            pltpu.VMEM((N_DEV - 1, mc, n), jnp.bfloat16),
            pltpu.SemaphoreType.DMA((N_DEV - 1, NSUB)),
            pltpu.SemaphoreType.DMA((N_DEV - 1, NSUB)),
            pltpu.SemaphoreType.DMA((N_DEV - 1, NSUB)),
            pltpu.SemaphoreType.DMA((N_DEV - 1, NSUB)),
            pltpu.SemaphoreType.DMA((N_DEV - 1, NSUB)),
            pltpu.SemaphoreType.DMA((N_DEV - 1, NSUB)),
            pltpu.SemaphoreType.DMA((N_DEV - 1, NSUB)),
            pltpu.SemaphoreType.DMA((N_DEV - 1, NSUB)),
        ],
        compiler_params=pltpu.CompilerParams(collective_id=0),
    )(xr)
    return out.reshape(m, n).astype(jnp.float32)
